# baseline (speedup 1.0000x reference)
import functools

import jax
import jax.numpy as jnp
from jax import lax
from jax.experimental import pallas as pl
from jax.experimental.pallas import tpu as pltpu
from jax.experimental.pallas import tpu_sc as plsc

EMBED_DIM = 64
NC = 2
NS = 16
NW = NC * NS

_C = 512
_IB = 128


@functools.partial(jax.jit, static_argnums=(2, 3))
def _gather(table, idx2d, V, B):
    b_per_w = B // NW
    n_chunks = b_per_w // _C
    rows_per_chunk = _C // _IB

    mesh = plsc.VectorSubcoreMesh(core_axis_name="c", subcore_axis_name="s")

    @functools.partial(
        pl.kernel,
        mesh=mesh,
        out_type=jax.ShapeDtypeStruct((B, EMBED_DIM), jnp.float32),
        scratch_types=[
            pltpu.VMEM((rows_per_chunk, _IB), jnp.int32),
            pltpu.VMEM((_C, EMBED_DIM), jnp.float32),
            pltpu.SemaphoreType.DMA,
        ],
        compiler_params=pltpu.CompilerParams(use_tc_tiling_on_sc=False),
    )
    def gather_kernel(table_hbm, idx_hbm, out_hbm, idx_v, rows_v, sem):
        wid = lax.axis_index("s") * NC + lax.axis_index("c")
        base = wid * b_per_w
        base_row = wid * (b_per_w // _IB)

        def body(i, carry):
            row0 = base_row + i * rows_per_chunk
            pltpu.sync_copy(idx_hbm.at[pl.ds(row0, rows_per_chunk)], idx_v)
            copies = []
            for j in range(rows_per_chunk):
                copies.append(
                    pltpu.async_copy(
                        table_hbm.at[idx_v.at[j]],
                        rows_v.at[pl.ds(j * _IB, _IB)],
                        sem,
                    )
                )
            for c in copies:
                c.wait()
            start = pl.multiple_of(base + i * _C, _C)
            pltpu.sync_copy(rows_v, out_hbm.at[pl.ds(start, _C)])
            return carry

        lax.fori_loop(0, n_chunks, body, 0)

    return gather_kernel(table, idx2d)


def kernel(pre_embedding, preembed_mask, embed_table):
    N, S = pre_embedding.shape
    B = N * S
    V = embed_table.shape[0]
    idx2d = pre_embedding.reshape(B // _IB, _IB)
    out = _gather(embed_table, idx2d, V, B)
    return out.reshape(N, S, EMBED_DIM), preembed_mask


# speedup vs baseline: 1.0478x; 1.0478x over previous
import functools

import jax
import jax.numpy as jnp
from jax import lax
from jax.experimental import pallas as pl
from jax.experimental.pallas import tpu as pltpu
from jax.experimental.pallas import tpu_sc as plsc

EMBED_DIM = 64
NC = 2
NS = 16
NW = NC * NS

_C = 512
_IB = 128
_NBUF = 2


@functools.partial(jax.jit, static_argnums=(2, 3))
def _gather(table, idx2d, V, B):
    b_per_w = B // NW
    n_chunks = b_per_w // _C
    n_outer = n_chunks // _NBUF
    R = _C // _IB
    idx_rows_w = b_per_w // _IB

    mesh = plsc.VectorSubcoreMesh(core_axis_name="c", subcore_axis_name="s")

    @functools.partial(
        pl.kernel,
        mesh=mesh,
        out_type=jax.ShapeDtypeStruct((B, EMBED_DIM), jnp.float32),
        scratch_types=[
            pltpu.VMEM((idx_rows_w, _IB), jnp.int32),
            pltpu.VMEM((_NBUF, _C, EMBED_DIM), jnp.float32),
            pltpu.SemaphoreType.DMA((_NBUF,)),
            pltpu.SemaphoreType.DMA((_NBUF,)),
        ],
        compiler_params=pltpu.CompilerParams(use_tc_tiling_on_sc=False),
    )
    def gather_kernel(table_hbm, idx_hbm, out_hbm, idx_all, rows_v, gsem, ssem):
        wid = lax.axis_index("s") * NC + lax.axis_index("c")
        base = wid * b_per_w
        pltpu.sync_copy(idx_hbm.at[pl.ds(wid * idx_rows_w, idx_rows_w)], idx_all)

        def fire_gather(i, b):
            for j in range(R):
                pltpu.async_copy(
                    table_hbm.at[idx_all.at[i * R + j]],
                    rows_v.at[b].at[pl.ds(j * _IB, _IB)],
                    gsem.at[b],
                )

        def wait_gather(b):
            pltpu.make_async_copy(
                table_hbm.at[pl.ds(0, _C)], rows_v.at[b], gsem.at[b]
            ).wait()

        def fire_store(i, b):
            start = pl.multiple_of(base + i * _C, _C)
            pltpu.async_copy(rows_v.at[b], out_hbm.at[pl.ds(start, _C)], ssem.at[b])

        def wait_store(b):
            pltpu.make_async_copy(
                table_hbm.at[pl.ds(0, _C)], rows_v.at[b], ssem.at[b]
            ).wait()

        def outer(o, carry):
            for b in range(_NBUF):
                i = o * _NBUF + b

                @pl.when(o > 0)
                def _():
                    wait_store(b)

                fire_gather(i, b)
                pb = (b - 1) % _NBUF
                if b == 0:
                    @pl.when(o > 0)
                    def _():
                        wait_gather(pb)
                        fire_store(i - 1, pb)
                else:
                    wait_gather(pb)
                    fire_store(i - 1, pb)
            return carry

        lax.fori_loop(0, n_outer, outer, 0)
        last = n_chunks - 1
        wait_gather(_NBUF - 1)
        fire_store(last, _NBUF - 1)
        for b in range(_NBUF):
            wait_store(b)

    return gather_kernel(table, idx2d)


def kernel(pre_embedding, preembed_mask, embed_table):
    N, S = pre_embedding.shape
    B = N * S
    V = embed_table.shape[0]
    idx2d = pre_embedding.reshape(B // _IB, _IB)
    out = _gather(embed_table, idx2d, V, B)
    return out.reshape(N, S, EMBED_DIM), preembed_mask
